# Initial kernel scaffold; baseline (speedup 1.0000x reference)
import jax
import jax.numpy as jnp
from jax.experimental import pallas as pl


def kernel(x, edge_index, edge_attr, batch, W_enc, b_enc, W_gcn, b_gcn, bn_gamma, bn_beta, W1, b1, W2, b2, W3, b3):
    raise NotImplementedError("write your pallas kernel here")


# speedup vs baseline: 7.2646x; 7.2646x over previous
import functools

import jax
import jax.numpy as jnp
from jax import lax
from jax.experimental import pallas as pl
from jax.experimental.pallas import tpu as pltpu
from jax.experimental.pallas import tpu_sc as plsc

N_NODES = 10000
N_EDGES = 160000
IN_DIM = 14
HID = 256
HALF = 128
OUT_DIM = 21
BN_EPS = 1e-5
N_ITERS = 8
SMOOTH = 0.5

ROWS_BLK = 400
N_BLKS = N_NODES // ROWS_BLK
CHUNK = 128
E_PER_SUB = N_EDGES // 16
N_CHUNKS = E_PER_SUB // CHUNK
TAIL = E_PER_SUB - N_CHUNKS * CHUNK
NODE_PAD = 10240
DEG_PER_SUB = NODE_PAD // 16

_sc_mesh = plsc.VectorSubcoreMesh(core_axis_name="c", subcore_axis_name="s")


@functools.partial(
    pl.kernel,
    out_type=jax.ShapeDtypeStruct((NODE_PAD,), jnp.float32),
    mesh=_sc_mesh,
    scratch_types=[
        pltpu.VMEM_SHARED((NODE_PAD,), jnp.float32),
        pltpu.VMEM((CHUNK,), jnp.int32),
        pltpu.VMEM((CHUNK,), jnp.float32),
        pltpu.VMEM((TAIL,), jnp.int32),
        pltpu.VMEM((TAIL,), jnp.float32),
        pltpu.VMEM((DEG_PER_SUB,), jnp.float32),
    ],
)
def _deg_kernel(dst_hbm, deg_hbm, acc_sh, dst_v, ones_v, dst_t, ones_t, buf_v):
    c = lax.axis_index("c")
    s = lax.axis_index("s")

    @pl.when(c == 0)
    def _():
        for t in range(DEG_PER_SUB // 16):
            buf_v[pl.ds(t * 16, 16)] = jnp.zeros((16,), jnp.float32)
        pltpu.sync_copy(buf_v, acc_sh.at[pl.ds(s * DEG_PER_SUB, DEG_PER_SUB)])
        for t in range(CHUNK // 16):
            ones_v[pl.ds(t * 16, 16)] = jnp.full((16,), 1.0, jnp.float32)
        ones_t[...] = jnp.full((TAIL,), 1.0, jnp.float32)
        plsc.subcore_barrier()

        base0 = s * E_PER_SUB

        def body(j, carry):
            base = base0 + j * CHUNK
            pltpu.sync_copy(dst_hbm.at[pl.ds(base, CHUNK)], dst_v)
            pltpu.sync_copy(ones_v, acc_sh.at[dst_v], add=True)
            return carry

        lax.fori_loop(0, N_CHUNKS, body, 0)
        pltpu.sync_copy(dst_hbm.at[pl.ds(base0 + N_CHUNKS * CHUNK, TAIL)], dst_t)
        pltpu.sync_copy(ones_t, acc_sh.at[dst_t], add=True)
        plsc.subcore_barrier()

        pltpu.sync_copy(acc_sh.at[pl.ds(s * DEG_PER_SUB, DEG_PER_SUB)], buf_v)
        for t in range(DEG_PER_SUB // 16):
            sl = pl.ds(t * 16, 16)
            buf_v[sl] = buf_v[sl] + 1.0
        pltpu.sync_copy(buf_v, deg_hbm.at[pl.ds(s * DEG_PER_SUB, DEG_PER_SUB)])


_SUB_STRIDE = 624
_SUB_SPAN = 640

@functools.partial(
    pl.kernel,
    out_type=jax.ShapeDtypeStruct((2 * N_NODES, HALF), jnp.float32),
    mesh=_sc_mesh,
    scratch_types=[
        pltpu.VMEM_SHARED((N_NODES, HALF), jnp.float32),
        pltpu.VMEM((CHUNK,), jnp.int32),
        pltpu.VMEM((CHUNK,), jnp.int32),
        pltpu.VMEM((CHUNK,), jnp.int32),
        pltpu.VMEM((CHUNK, HALF), jnp.float32),
        pltpu.VMEM((TAIL,), jnp.int32),
        pltpu.VMEM((TAIL,), jnp.int32),
        pltpu.VMEM((TAIL,), jnp.int32),
        pltpu.VMEM((TAIL, HALF), jnp.float32),
        pltpu.VMEM((32, HALF), jnp.float32),
        pltpu.SemaphoreType.DMA,
    ],
)
def _agg_kernel(hws_hbm, src_hbm, dst_hbm, out_hbm,
                acc_sh, src_v, dst_v, idx_v, rows_v,
                src_t, dst_t, idxt_v, rows_t, zero_v, sem):
    c = lax.axis_index("c")
    s = lax.axis_index("s")
    coff = c * N_NODES

    for t in range(32 * (HALF // 16)):
        r, q = divmod(t, HALF // 16)
        zero_v[r, pl.ds(q * 16, 16)] = jnp.zeros((16,), jnp.float32)

    def zbody(k, carry):
        pltpu.sync_copy(zero_v, acc_sh.at[pl.ds(s * _SUB_STRIDE + k * 32, 32)])
        return carry

    lax.fori_loop(0, _SUB_SPAN // 32, zbody, 0)
    plsc.subcore_barrier()

    base0 = s * E_PER_SUB

    def body(j, carry):
        base = base0 + j * CHUNK
        pltpu.sync_copy(src_hbm.at[pl.ds(base, CHUNK)], src_v)
        pltpu.sync_copy(dst_hbm.at[pl.ds(base, CHUNK)], dst_v)
        for t in range(CHUNK // 16):
            sl = pl.ds(t * 16, 16)
            idx_v[sl] = src_v[sl] + coff
        pltpu.async_copy(hws_hbm.at[idx_v], rows_v, sem).wait()
        pltpu.sync_copy(rows_v, acc_sh.at[dst_v], add=True)
        return carry

    lax.fori_loop(0, N_CHUNKS, body, 0)

    baset = base0 + N_CHUNKS * CHUNK
    pltpu.sync_copy(src_hbm.at[pl.ds(baset, TAIL)], src_t)
    pltpu.sync_copy(dst_hbm.at[pl.ds(baset, TAIL)], dst_t)
    idxt_v[...] = src_t[...] + coff
    pltpu.async_copy(hws_hbm.at[idxt_v], rows_t, sem).wait()
    pltpu.sync_copy(rows_t, acc_sh.at[dst_t], add=True)
    plsc.subcore_barrier()

    def wbody(k, carry):
        nbase = s * _SUB_STRIDE + k * CHUNK
        pltpu.sync_copy(acc_sh.at[pl.ds(nbase, CHUNK)], rows_v)
        pltpu.sync_copy(rows_v, out_hbm.at[pl.ds(coff + nbase, CHUNK)])
        return carry

    lax.fori_loop(0, _SUB_SPAN // CHUNK, wbody, 0)


def _full(shape):
    return pl.BlockSpec(shape, lambda i: tuple(0 for _ in shape))


def _enc_body(x_ref, we_ref, be_ref, deg_ref, h_ref, dinv_ref):
    dinv_ref[...] = lax.rsqrt(deg_ref[...])
    h_ref[...] = (
        jnp.dot(x_ref[...], we_ref[...], preferred_element_type=jnp.float32)
        + be_ref[...]
    )


def _enc_call(x, W_enc, b_enc2d, deg2d):
    return pl.pallas_call(
        _enc_body,
        grid=(N_BLKS,),
        in_specs=[
            pl.BlockSpec((ROWS_BLK, IN_DIM), lambda i: (i, 0)),
            _full((IN_DIM, HID)),
            _full((1, HID)),
            pl.BlockSpec((1, 1, ROWS_BLK), lambda i: (i, 0, 0)),
        ],
        out_specs=[
            pl.BlockSpec((ROWS_BLK, HID), lambda i: (i, 0)),
            pl.BlockSpec((1, 1, ROWS_BLK), lambda i: (i, 0, 0)),
        ],
        out_shape=[
            jax.ShapeDtypeStruct((N_NODES, HID), jnp.float32),
            jax.ShapeDtypeStruct((N_BLKS, 1, ROWS_BLK), jnp.float32),
        ],
    )(x, W_enc, b_enc2d, deg2d)


def _scale_body(h_ref, dinv_ref, wg_ref, cvec_ref, hws_ref):
    wp = wg_ref[...] * cvec_ref[...]
    hw = jnp.dot(h_ref[...], wp, preferred_element_type=jnp.float32)
    d = dinv_ref[...]
    hws_ref[0] = hw[:, :HALF] * d
    hws_ref[1] = hw[:, HALF:] * d


def _scale_call(h, dinv_b, W_gcn, cvec2d):
    return pl.pallas_call(
        _scale_body,
        grid=(N_BLKS,),
        in_specs=[
            pl.BlockSpec((ROWS_BLK, HID), lambda i: (i, 0)),
            pl.BlockSpec((ROWS_BLK, HALF), lambda i: (i, 0)),
            _full((HID, HID)),
            _full((1, HID)),
        ],
        out_specs=pl.BlockSpec((2, ROWS_BLK, HALF), lambda i: (0, i, 0)),
        out_shape=jax.ShapeDtypeStruct((2, N_NODES, HALF), jnp.float32),
    )(h, dinv_b, W_gcn, cvec2d)


def _smooth(h_ref, acc_ref, hwsp_ref, dinv_ref, cvec_ref, bg_ref, bb_ref):
    d = dinv_ref[...]
    a = acc_ref[...]
    p = hwsp_ref[...]
    g0 = (a[0] + p[0]) * d
    g1 = (a[1] + p[1]) * d
    bpp = bg_ref[...] * cvec_ref[...] + bb_ref[...]
    g = jnp.concatenate([g0, g1], axis=1) + bpp
    g = jnp.maximum(g, 0.0)
    return SMOOTH * h_ref[...] + (1.0 - SMOOTH) * g


def _iter_body(h_ref, acc_ref, hwsp_ref, dinv_ref, wg_ref, cvec_ref,
               bg_ref, bb_ref, hn_ref, hws_ref):
    hn = _smooth(h_ref, acc_ref, hwsp_ref, dinv_ref, cvec_ref, bg_ref, bb_ref)
    hn_ref[...] = hn
    wp = wg_ref[...] * cvec_ref[...]
    hw = jnp.dot(hn, wp, preferred_element_type=jnp.float32)
    d = dinv_ref[...]
    hws_ref[0] = hw[:, :HALF] * d
    hws_ref[1] = hw[:, HALF:] * d


def _iter_call(h, acc3, hws3, dinv_b, W_gcn, cvec2d, bg2d, bb2d):
    return pl.pallas_call(
        _iter_body,
        grid=(N_BLKS,),
        in_specs=[
            pl.BlockSpec((ROWS_BLK, HID), lambda i: (i, 0)),
            pl.BlockSpec((2, ROWS_BLK, HALF), lambda i: (0, i, 0)),
            pl.BlockSpec((2, ROWS_BLK, HALF), lambda i: (0, i, 0)),
            pl.BlockSpec((ROWS_BLK, HALF), lambda i: (i, 0)),
            _full((HID, HID)),
            _full((1, HID)),
            _full((1, HID)),
            _full((1, HID)),
        ],
        out_specs=[
            pl.BlockSpec((ROWS_BLK, HID), lambda i: (i, 0)),
            pl.BlockSpec((2, ROWS_BLK, HALF), lambda i: (0, i, 0)),
        ],
        out_shape=[
            jax.ShapeDtypeStruct((N_NODES, HID), jnp.float32),
            jax.ShapeDtypeStruct((2, N_NODES, HALF), jnp.float32),
        ],
    )(h, acc3, hws3, dinv_b, W_gcn, cvec2d, bg2d, bb2d)


def _head_body(h_ref, acc_ref, hwsp_ref, dinv_ref, cvec_ref, bg_ref, bb_ref,
               w1_ref, b1_ref, w2_ref, b2_ref, w3_ref, b3_ref, out_ref):
    hn = _smooth(h_ref, acc_ref, hwsp_ref, dinv_ref, cvec_ref, bg_ref, bb_ref)
    t = jnp.maximum(
        jnp.dot(hn, w1_ref[...], preferred_element_type=jnp.float32)
        + b1_ref[...], 0.0)
    t = jnp.maximum(
        jnp.dot(t, w2_ref[...], preferred_element_type=jnp.float32)
        + b2_ref[...], 0.0)
    out_ref[...] = (
        jnp.dot(t, w3_ref[...], preferred_element_type=jnp.float32)
        + b3_ref[...])


def _head_call(h, acc3, hws3, dinv_b, cvec2d, bg2d, bb2d,
               W1, b1_2d, W2, b2_2d, W3, b3_2d):
    return pl.pallas_call(
        _head_body,
        grid=(N_BLKS,),
        in_specs=[
            pl.BlockSpec((ROWS_BLK, HID), lambda i: (i, 0)),
            pl.BlockSpec((2, ROWS_BLK, HALF), lambda i: (0, i, 0)),
            pl.BlockSpec((2, ROWS_BLK, HALF), lambda i: (0, i, 0)),
            pl.BlockSpec((ROWS_BLK, HALF), lambda i: (i, 0)),
            _full((1, HID)),
            _full((1, HID)),
            _full((1, HID)),
            _full((HID, HID)),
            _full((1, HID)),
            _full((HID, HID)),
            _full((1, HID)),
            _full((HID, OUT_DIM)),
            _full((1, OUT_DIM)),
        ],
        out_specs=pl.BlockSpec((ROWS_BLK, OUT_DIM), lambda i: (i, 0)),
        out_shape=jax.ShapeDtypeStruct((N_NODES, OUT_DIM), jnp.float32),
    )(h, acc3, hws3, dinv_b, cvec2d, bg2d, bb2d, W1, b1_2d, W2, b2_2d, W3, b3_2d)


def kernel(x, edge_index, edge_attr, batch, W_enc, b_enc, W_gcn, b_gcn,
           bn_gamma, bn_beta, W1, b1, W2, b2, W3, b3):
    del edge_attr, batch
    src = edge_index[0]
    dst = edge_index[1]

    inv_std = 1.0 / (1.0 + BN_EPS) ** 0.5
    b_enc2d = b_enc.reshape(1, HID)
    cvec2d = (bn_gamma * inv_std).reshape(1, HID)
    bg2d = b_gcn.reshape(1, HID)
    bb2d = bn_beta.reshape(1, HID)
    b1_2d = b1.reshape(1, HID)
    b2_2d = b2.reshape(1, HID)
    b3_2d = b3.reshape(1, OUT_DIM)

    deg = _deg_kernel(dst)
    deg2d = deg[:N_NODES].reshape(N_BLKS, 1, ROWS_BLK)
    h, dinv2d = _enc_call(x, W_enc, b_enc2d, deg2d)
    dinv_b = jnp.broadcast_to(
        dinv2d.reshape(N_NODES, 1), (N_NODES, HALF))
    hws3 = _scale_call(h, dinv_b, W_gcn, cvec2d)

    for it in range(N_ITERS):
        acc = _agg_kernel(hws3.reshape(2 * N_NODES, HALF), src, dst)
        acc3 = acc.reshape(2, N_NODES, HALF)
        if it < N_ITERS - 1:
            h, hws3 = _iter_call(h, acc3, hws3, dinv_b, W_gcn, cvec2d,
                                 bg2d, bb2d)
        else:
            out = _head_call(h, acc3, hws3, dinv_b, cvec2d, bg2d, bb2d,
                             W1, b1_2d, W2, b2_2d, W3, b3_2d)
    return out


# speedup vs baseline: 10.8476x; 1.4932x over previous
import functools

import jax
import jax.numpy as jnp
from jax import lax
from jax.experimental import pallas as pl
from jax.experimental.pallas import tpu as pltpu
from jax.experimental.pallas import tpu_sc as plsc

N_NODES = 10000
N_EDGES = 160000
IN_DIM = 14
HID = 256
HALF = 128
OUT_DIM = 21
BN_EPS = 1e-5
N_ITERS = 8
SMOOTH = 0.5

ROWS_BLK = 400
N_BLKS = N_NODES // ROWS_BLK
CHUNK = 128
E_PER_SUB = N_EDGES // 16
N_CHUNKS = E_PER_SUB // CHUNK
TAIL = E_PER_SUB - N_CHUNKS * CHUNK
NODE_PAD = 10240
DEG_PER_SUB = NODE_PAD // 16

_sc_mesh = plsc.VectorSubcoreMesh(core_axis_name="c", subcore_axis_name="s")


@functools.partial(
    pl.kernel,
    out_type=jax.ShapeDtypeStruct((NODE_PAD,), jnp.float32),
    mesh=_sc_mesh,
    scratch_types=[
        pltpu.VMEM_SHARED((NODE_PAD,), jnp.float32),
        pltpu.VMEM((CHUNK,), jnp.int32),
        pltpu.VMEM((CHUNK,), jnp.float32),
        pltpu.VMEM((TAIL,), jnp.int32),
        pltpu.VMEM((TAIL,), jnp.float32),
        pltpu.VMEM((DEG_PER_SUB,), jnp.float32),
    ],
)
def _deg_kernel(dst_hbm, deg_hbm, acc_sh, dst_v, ones_v, dst_t, ones_t, buf_v):
    c = lax.axis_index("c")
    s = lax.axis_index("s")

    @pl.when(c == 0)
    def _():
        for t in range(DEG_PER_SUB // 16):
            buf_v[pl.ds(t * 16, 16)] = jnp.zeros((16,), jnp.float32)
        pltpu.sync_copy(buf_v, acc_sh.at[pl.ds(s * DEG_PER_SUB, DEG_PER_SUB)])
        for t in range(CHUNK // 16):
            ones_v[pl.ds(t * 16, 16)] = jnp.full((16,), 1.0, jnp.float32)
        ones_t[...] = jnp.full((TAIL,), 1.0, jnp.float32)
        plsc.subcore_barrier()

        base0 = s * E_PER_SUB

        def body(j, carry):
            base = base0 + j * CHUNK
            pltpu.sync_copy(dst_hbm.at[pl.ds(base, CHUNK)], dst_v)
            pltpu.sync_copy(ones_v, acc_sh.at[dst_v], add=True)
            return carry

        lax.fori_loop(0, N_CHUNKS, body, 0)
        pltpu.sync_copy(dst_hbm.at[pl.ds(base0 + N_CHUNKS * CHUNK, TAIL)], dst_t)
        pltpu.sync_copy(ones_t, acc_sh.at[dst_t], add=True)
        plsc.subcore_barrier()

        pltpu.sync_copy(acc_sh.at[pl.ds(s * DEG_PER_SUB, DEG_PER_SUB)], buf_v)
        for t in range(DEG_PER_SUB // 16):
            sl = pl.ds(t * 16, 16)
            buf_v[sl] = buf_v[sl] + 1.0
        pltpu.sync_copy(buf_v, deg_hbm.at[pl.ds(s * DEG_PER_SUB, DEG_PER_SUB)])


_SUB_STRIDE = 624
_SUB_SPAN = 640

@functools.partial(
    pl.kernel,
    out_type=jax.ShapeDtypeStruct((2 * N_NODES, HALF), jnp.float32),
    mesh=_sc_mesh,
    scratch_types=[
        pltpu.VMEM_SHARED((N_NODES, HALF), jnp.float32),
        pltpu.VMEM((CHUNK,), jnp.int32),
        pltpu.VMEM((CHUNK,), jnp.int32),
        pltpu.VMEM((CHUNK,), jnp.int32),
        pltpu.VMEM((CHUNK,), jnp.int32),
        pltpu.VMEM((CHUNK,), jnp.int32),
        pltpu.VMEM((CHUNK,), jnp.int32),
        pltpu.VMEM((CHUNK, HALF), jnp.float32),
        pltpu.VMEM((CHUNK, HALF), jnp.float32),
        pltpu.SemaphoreType.DMA,
        pltpu.SemaphoreType.DMA,
        pltpu.VMEM((TAIL,), jnp.int32),
        pltpu.VMEM((TAIL,), jnp.int32),
        pltpu.VMEM((TAIL,), jnp.int32),
        pltpu.VMEM((TAIL, HALF), jnp.float32),
        pltpu.VMEM((32, HALF), jnp.float32),
        pltpu.SemaphoreType.DMA,
    ],
)
def _agg_kernel(hws_hbm, src_hbm, dst_hbm, out_hbm,
                acc_sh, src0, src1, dst0, dst1, idx0, idx1, rows0, rows1,
                sem0, sem1, src_t, dst_t, idxt_v, rows_t, zero_v, sem):
    c = lax.axis_index("c")
    s = lax.axis_index("s")
    coff = c * N_NODES
    src_b = (src0, src1)
    dst_b = (dst0, dst1)
    idx_b = (idx0, idx1)
    rows_b = (rows0, rows1)
    sem_b = (sem0, sem1)

    for t in range(32 * (HALF // 16)):
        r, q = divmod(t, HALF // 16)
        zero_v[r, pl.ds(q * 16, 16)] = jnp.zeros((16,), jnp.float32)

    def zbody(k, carry):
        pltpu.sync_copy(zero_v, acc_sh.at[pl.ds(s * _SUB_STRIDE + k * 32, 32)])
        return carry

    lax.fori_loop(0, _SUB_SPAN // 32, zbody, 0)
    plsc.subcore_barrier()

    base0 = s * E_PER_SUB

    def prefetch(b, j):
        base = base0 + j * CHUNK
        pltpu.sync_copy(src_hbm.at[pl.ds(base, CHUNK)], src_b[b])
        pltpu.sync_copy(dst_hbm.at[pl.ds(base, CHUNK)], dst_b[b])
        for t in range(CHUNK // 16):
            sl = pl.ds(t * 16, 16)
            idx_b[b][sl] = src_b[b][sl] + coff
        pltpu.async_copy(hws_hbm.at[idx_b[b]], rows_b[b], sem_b[b])

    prefetch(0, 0)
    prefetch(1, 1)

    def body(k, carry):
        for b in range(2):
            j = 2 * k + b
            pltpu.make_async_copy(hws_hbm.at[idx_b[b]], rows_b[b],
                                  sem_b[b]).wait()
            pltpu.sync_copy(rows_b[b], acc_sh.at[dst_b[b]], add=True)

            @pl.when(j + 2 < N_CHUNKS)
            def _():
                prefetch(b, j + 2)

        return carry

    lax.fori_loop(0, N_CHUNKS // 2, body, 0)

    baset = base0 + N_CHUNKS * CHUNK
    pltpu.sync_copy(src_hbm.at[pl.ds(baset, TAIL)], src_t)
    pltpu.sync_copy(dst_hbm.at[pl.ds(baset, TAIL)], dst_t)
    idxt_v[...] = src_t[...] + coff
    pltpu.async_copy(hws_hbm.at[idxt_v], rows_t, sem).wait()
    pltpu.sync_copy(rows_t, acc_sh.at[dst_t], add=True)
    plsc.subcore_barrier()

    def wbody(k, carry):
        nbase = s * _SUB_STRIDE + k * CHUNK
        pltpu.sync_copy(acc_sh.at[pl.ds(nbase, CHUNK)], rows0)
        pltpu.sync_copy(rows0, out_hbm.at[pl.ds(coff + nbase, CHUNK)])
        return carry

    lax.fori_loop(0, _SUB_SPAN // CHUNK, wbody, 0)


def _full(shape):
    return pl.BlockSpec(shape, lambda i: tuple(0 for _ in shape))


def _enc_body(x_ref, we_ref, be_ref, deg_ref, h_ref, dinv_ref):
    dinv_ref[...] = lax.rsqrt(deg_ref[...])
    h_ref[...] = (
        jnp.dot(x_ref[...], we_ref[...], preferred_element_type=jnp.float32)
        + be_ref[...]
    )


def _enc_call(x, W_enc, b_enc2d, deg2d):
    return pl.pallas_call(
        _enc_body,
        grid=(N_BLKS,),
        in_specs=[
            pl.BlockSpec((ROWS_BLK, IN_DIM), lambda i: (i, 0)),
            _full((IN_DIM, HID)),
            _full((1, HID)),
            pl.BlockSpec((1, 1, ROWS_BLK), lambda i: (i, 0, 0)),
        ],
        out_specs=[
            pl.BlockSpec((ROWS_BLK, HID), lambda i: (i, 0)),
            pl.BlockSpec((1, 1, ROWS_BLK), lambda i: (i, 0, 0)),
        ],
        out_shape=[
            jax.ShapeDtypeStruct((N_NODES, HID), jnp.float32),
            jax.ShapeDtypeStruct((N_BLKS, 1, ROWS_BLK), jnp.float32),
        ],
    )(x, W_enc, b_enc2d, deg2d)


def _scale_body(h_ref, dinv_ref, wg_ref, cvec_ref, hws_ref):
    wp = wg_ref[...] * cvec_ref[...]
    hw = jnp.dot(h_ref[...], wp, preferred_element_type=jnp.float32)
    d = dinv_ref[...]
    hws_ref[0] = hw[:, :HALF] * d
    hws_ref[1] = hw[:, HALF:] * d


def _scale_call(h, dinv_b, W_gcn, cvec2d):
    return pl.pallas_call(
        _scale_body,
        grid=(N_BLKS,),
        in_specs=[
            pl.BlockSpec((ROWS_BLK, HID), lambda i: (i, 0)),
            pl.BlockSpec((ROWS_BLK, HALF), lambda i: (i, 0)),
            _full((HID, HID)),
            _full((1, HID)),
        ],
        out_specs=pl.BlockSpec((2, ROWS_BLK, HALF), lambda i: (0, i, 0)),
        out_shape=jax.ShapeDtypeStruct((2, N_NODES, HALF), jnp.float32),
    )(h, dinv_b, W_gcn, cvec2d)


def _smooth(h_ref, acc_ref, hwsp_ref, dinv_ref, cvec_ref, bg_ref, bb_ref):
    d = dinv_ref[...]
    a = acc_ref[...]
    p = hwsp_ref[...]
    g0 = (a[0] + p[0]) * d
    g1 = (a[1] + p[1]) * d
    bpp = bg_ref[...] * cvec_ref[...] + bb_ref[...]
    g = jnp.concatenate([g0, g1], axis=1) + bpp
    g = jnp.maximum(g, 0.0)
    return SMOOTH * h_ref[...] + (1.0 - SMOOTH) * g


def _iter_body(h_ref, acc_ref, hwsp_ref, dinv_ref, wg_ref, cvec_ref,
               bg_ref, bb_ref, hn_ref, hws_ref):
    hn = _smooth(h_ref, acc_ref, hwsp_ref, dinv_ref, cvec_ref, bg_ref, bb_ref)
    hn_ref[...] = hn
    wp = wg_ref[...] * cvec_ref[...]
    hw = jnp.dot(hn, wp, preferred_element_type=jnp.float32)
    d = dinv_ref[...]
    hws_ref[0] = hw[:, :HALF] * d
    hws_ref[1] = hw[:, HALF:] * d


def _iter_call(h, acc3, hws3, dinv_b, W_gcn, cvec2d, bg2d, bb2d):
    return pl.pallas_call(
        _iter_body,
        grid=(N_BLKS,),
        in_specs=[
            pl.BlockSpec((ROWS_BLK, HID), lambda i: (i, 0)),
            pl.BlockSpec((2, ROWS_BLK, HALF), lambda i: (0, i, 0)),
            pl.BlockSpec((2, ROWS_BLK, HALF), lambda i: (0, i, 0)),
            pl.BlockSpec((ROWS_BLK, HALF), lambda i: (i, 0)),
            _full((HID, HID)),
            _full((1, HID)),
            _full((1, HID)),
            _full((1, HID)),
        ],
        out_specs=[
            pl.BlockSpec((ROWS_BLK, HID), lambda i: (i, 0)),
            pl.BlockSpec((2, ROWS_BLK, HALF), lambda i: (0, i, 0)),
        ],
        out_shape=[
            jax.ShapeDtypeStruct((N_NODES, HID), jnp.float32),
            jax.ShapeDtypeStruct((2, N_NODES, HALF), jnp.float32),
        ],
    )(h, acc3, hws3, dinv_b, W_gcn, cvec2d, bg2d, bb2d)


def _head_body(h_ref, acc_ref, hwsp_ref, dinv_ref, cvec_ref, bg_ref, bb_ref,
               w1_ref, b1_ref, w2_ref, b2_ref, w3_ref, b3_ref, out_ref):
    hn = _smooth(h_ref, acc_ref, hwsp_ref, dinv_ref, cvec_ref, bg_ref, bb_ref)
    t = jnp.maximum(
        jnp.dot(hn, w1_ref[...], preferred_element_type=jnp.float32)
        + b1_ref[...], 0.0)
    t = jnp.maximum(
        jnp.dot(t, w2_ref[...], preferred_element_type=jnp.float32)
        + b2_ref[...], 0.0)
    out_ref[...] = (
        jnp.dot(t, w3_ref[...], preferred_element_type=jnp.float32)
        + b3_ref[...])


def _head_call(h, acc3, hws3, dinv_b, cvec2d, bg2d, bb2d,
               W1, b1_2d, W2, b2_2d, W3, b3_2d):
    return pl.pallas_call(
        _head_body,
        grid=(N_BLKS,),
        in_specs=[
            pl.BlockSpec((ROWS_BLK, HID), lambda i: (i, 0)),
            pl.BlockSpec((2, ROWS_BLK, HALF), lambda i: (0, i, 0)),
            pl.BlockSpec((2, ROWS_BLK, HALF), lambda i: (0, i, 0)),
            pl.BlockSpec((ROWS_BLK, HALF), lambda i: (i, 0)),
            _full((1, HID)),
            _full((1, HID)),
            _full((1, HID)),
            _full((HID, HID)),
            _full((1, HID)),
            _full((HID, HID)),
            _full((1, HID)),
            _full((HID, OUT_DIM)),
            _full((1, OUT_DIM)),
        ],
        out_specs=pl.BlockSpec((ROWS_BLK, OUT_DIM), lambda i: (i, 0)),
        out_shape=jax.ShapeDtypeStruct((N_NODES, OUT_DIM), jnp.float32),
    )(h, acc3, hws3, dinv_b, cvec2d, bg2d, bb2d, W1, b1_2d, W2, b2_2d, W3, b3_2d)


def kernel(x, edge_index, edge_attr, batch, W_enc, b_enc, W_gcn, b_gcn,
           bn_gamma, bn_beta, W1, b1, W2, b2, W3, b3):
    del edge_attr, batch
    src = edge_index[0]
    dst = edge_index[1]

    inv_std = 1.0 / (1.0 + BN_EPS) ** 0.5
    b_enc2d = b_enc.reshape(1, HID)
    cvec2d = (bn_gamma * inv_std).reshape(1, HID)
    bg2d = b_gcn.reshape(1, HID)
    bb2d = bn_beta.reshape(1, HID)
    b1_2d = b1.reshape(1, HID)
    b2_2d = b2.reshape(1, HID)
    b3_2d = b3.reshape(1, OUT_DIM)

    deg = _deg_kernel(dst)
    deg2d = deg[:N_NODES].reshape(N_BLKS, 1, ROWS_BLK)
    h, dinv2d = _enc_call(x, W_enc, b_enc2d, deg2d)
    dinv_b = jnp.broadcast_to(
        dinv2d.reshape(N_NODES, 1), (N_NODES, HALF))
    hws3 = _scale_call(h, dinv_b, W_gcn, cvec2d)

    for it in range(N_ITERS):
        acc = _agg_kernel(hws3.reshape(2 * N_NODES, HALF), src, dst)
        acc3 = acc.reshape(2, N_NODES, HALF)
        if it < N_ITERS - 1:
            h, hws3 = _iter_call(h, acc3, hws3, dinv_b, W_gcn, cvec2d,
                                 bg2d, bb2d)
        else:
            out = _head_call(h, acc3, hws3, dinv_b, cvec2d, bg2d, bb2d,
                             W1, b1_2d, W2, b2_2d, W3, b3_2d)
    return out


# speedup vs baseline: 12.4505x; 1.1478x over previous
import functools

import jax
import jax.numpy as jnp
from jax import lax
from jax.experimental import pallas as pl
from jax.experimental.pallas import tpu as pltpu
from jax.experimental.pallas import tpu_sc as plsc

N_NODES = 10000
N_EDGES = 160000
IN_DIM = 14
HID = 256
HALF = 128
OUT_DIM = 21
BN_EPS = 1e-5
N_ITERS = 8
SMOOTH = 0.5

ROWS_BLK = 400
N_BLKS = N_NODES // ROWS_BLK
CHUNK = 128
E_PER_SUB = N_EDGES // 16
N_CHUNKS = E_PER_SUB // CHUNK
TAIL = E_PER_SUB - N_CHUNKS * CHUNK
NODE_PAD = 10240
DEG_PER_SUB = NODE_PAD // 16

_sc_mesh = plsc.VectorSubcoreMesh(core_axis_name="c", subcore_axis_name="s")


@functools.partial(
    pl.kernel,
    out_type=jax.ShapeDtypeStruct((NODE_PAD,), jnp.float32),
    mesh=_sc_mesh,
    scratch_types=[
        pltpu.VMEM_SHARED((NODE_PAD,), jnp.float32),
        pltpu.VMEM((CHUNK,), jnp.int32),
        pltpu.VMEM((CHUNK,), jnp.float32),
        pltpu.VMEM((TAIL,), jnp.int32),
        pltpu.VMEM((TAIL,), jnp.float32),
        pltpu.VMEM((DEG_PER_SUB,), jnp.float32),
    ],
)
def _deg_kernel(dst_hbm, deg_hbm, acc_sh, dst_v, ones_v, dst_t, ones_t, buf_v):
    c = lax.axis_index("c")
    s = lax.axis_index("s")

    @pl.when(c == 0)
    def _():
        for t in range(DEG_PER_SUB // 16):
            buf_v[pl.ds(t * 16, 16)] = jnp.zeros((16,), jnp.float32)
        pltpu.sync_copy(buf_v, acc_sh.at[pl.ds(s * DEG_PER_SUB, DEG_PER_SUB)])
        for t in range(CHUNK // 16):
            ones_v[pl.ds(t * 16, 16)] = jnp.full((16,), 1.0, jnp.float32)
        ones_t[...] = jnp.full((TAIL,), 1.0, jnp.float32)
        plsc.subcore_barrier()

        base0 = s * E_PER_SUB

        def body(j, carry):
            base = base0 + j * CHUNK
            pltpu.sync_copy(dst_hbm.at[pl.ds(base, CHUNK)], dst_v)
            pltpu.sync_copy(ones_v, acc_sh.at[dst_v], add=True)
            return carry

        lax.fori_loop(0, N_CHUNKS, body, 0)
        pltpu.sync_copy(dst_hbm.at[pl.ds(base0 + N_CHUNKS * CHUNK, TAIL)], dst_t)
        pltpu.sync_copy(ones_t, acc_sh.at[dst_t], add=True)
        plsc.subcore_barrier()

        pltpu.sync_copy(acc_sh.at[pl.ds(s * DEG_PER_SUB, DEG_PER_SUB)], buf_v)
        for t in range(DEG_PER_SUB // 16):
            sl = pl.ds(t * 16, 16)
            buf_v[sl] = buf_v[sl] + 1.0
        pltpu.sync_copy(buf_v, deg_hbm.at[pl.ds(s * DEG_PER_SUB, DEG_PER_SUB)])


_SUB_STRIDE = 624
_SUB_SPAN = 640

@functools.partial(
    pl.kernel,
    out_type=jax.ShapeDtypeStruct((2 * N_NODES, HALF), jnp.float32),
    mesh=_sc_mesh,
    scratch_types=[
        pltpu.VMEM_SHARED((N_NODES, HALF), jnp.float32),
        pltpu.VMEM((E_PER_SUB,), jnp.int32),
        pltpu.VMEM((CHUNK,), jnp.int32),
        pltpu.VMEM((CHUNK,), jnp.int32),
        pltpu.VMEM((CHUNK, HALF), jnp.float32),
        pltpu.VMEM((CHUNK, HALF), jnp.float32),
        pltpu.SemaphoreType.DMA,
        pltpu.SemaphoreType.DMA,
        pltpu.VMEM((TAIL,), jnp.int32),
        pltpu.VMEM((TAIL, HALF), jnp.float32),
        pltpu.VMEM((16, HALF), jnp.float32),
        pltpu.SemaphoreType.DMA,
    ],
)
def _agg_kernel(hws_hbm, src_hbm, dst_hbm, out_hbm,
                acc_sh, srcbig, dst0, dst1, rows0, rows1,
                sem0, sem1, dst_t, rows_t, zero_v, sem):
    c = lax.axis_index("c")
    s = lax.axis_index("s")
    coff = c * N_NODES
    dst_b = (dst0, dst1)
    rows_b = (rows0, rows1)
    sem_b = (sem0, sem1)

    for t in range(16 * (HALF // 16)):
        r, q = divmod(t, HALF // 16)
        zero_v[r, pl.ds(q * 16, 16)] = jnp.zeros((16,), jnp.float32)

    base0 = s * E_PER_SUB
    pltpu.sync_copy(src_hbm.at[pl.ds(base0, E_PER_SUB)], srcbig)

    def zbody(k, carry):
        pltpu.sync_copy(zero_v, acc_sh.at[pl.ds(s * _SUB_STRIDE + k * 16, 16)])
        return carry

    lax.fori_loop(0, _SUB_SPAN // 16, zbody, 0)

    def cbody(k, carry):
        sl = pl.ds(k * 16, 16)
        srcbig[sl] = srcbig[sl] + coff
        return carry

    lax.fori_loop(0, E_PER_SUB // 16, cbody, 0)
    plsc.subcore_barrier()

    def prefetch(b, j):
        pltpu.sync_copy(dst_hbm.at[pl.ds(base0 + j * CHUNK, CHUNK)], dst_b[b])
        pltpu.async_copy(hws_hbm.at[srcbig.at[pl.ds(j * CHUNK, CHUNK)]],
                         rows_b[b], sem_b[b])

    prefetch(0, 0)
    prefetch(1, 1)

    def body(k, carry):
        for b in range(2):
            j = 2 * k + b
            pltpu.make_async_copy(
                hws_hbm.at[srcbig.at[pl.ds(j * CHUNK, CHUNK)]],
                rows_b[b], sem_b[b]).wait()
            pltpu.sync_copy(rows_b[b], acc_sh.at[dst_b[b]], add=True)

            @pl.when(j + 2 < N_CHUNKS)
            def _():
                prefetch(b, j + 2)

        return carry

    lax.fori_loop(0, N_CHUNKS // 2, body, 0)

    baset = N_CHUNKS * CHUNK
    pltpu.sync_copy(dst_hbm.at[pl.ds(base0 + baset, TAIL)], dst_t)
    pltpu.async_copy(hws_hbm.at[srcbig.at[pl.ds(baset, TAIL)]],
                     rows_t, sem).wait()
    pltpu.sync_copy(rows_t, acc_sh.at[dst_t], add=True)
    plsc.subcore_barrier()

    def wbody(k, carry):
        nbase = s * _SUB_STRIDE + k * CHUNK
        pltpu.sync_copy(acc_sh.at[pl.ds(nbase, CHUNK)], rows0)
        pltpu.sync_copy(rows0, out_hbm.at[pl.ds(coff + nbase, CHUNK)])
        return carry

    lax.fori_loop(0, _SUB_SPAN // CHUNK, wbody, 0)


def _full(shape):
    return pl.BlockSpec(shape, lambda i: tuple(0 for _ in shape))


def _enc_body(x_ref, we_ref, be_ref, deg_ref, h_ref, dinv_ref):
    dinv_ref[...] = lax.rsqrt(deg_ref[...])
    h_ref[...] = (
        jnp.dot(x_ref[...], we_ref[...], preferred_element_type=jnp.float32)
        + be_ref[...]
    )


def _enc_call(x, W_enc, b_enc2d, deg2d):
    return pl.pallas_call(
        _enc_body,
        grid=(N_BLKS,),
        in_specs=[
            pl.BlockSpec((ROWS_BLK, IN_DIM), lambda i: (i, 0)),
            _full((IN_DIM, HID)),
            _full((1, HID)),
            pl.BlockSpec((1, 1, ROWS_BLK), lambda i: (i, 0, 0)),
        ],
        out_specs=[
            pl.BlockSpec((ROWS_BLK, HID), lambda i: (i, 0)),
            pl.BlockSpec((1, 1, ROWS_BLK), lambda i: (i, 0, 0)),
        ],
        out_shape=[
            jax.ShapeDtypeStruct((N_NODES, HID), jnp.float32),
            jax.ShapeDtypeStruct((N_BLKS, 1, ROWS_BLK), jnp.float32),
        ],
    )(x, W_enc, b_enc2d, deg2d)


def _scale_body(h_ref, dinv_ref, wg_ref, cvec_ref, hws_ref):
    wp = wg_ref[...] * cvec_ref[...]
    hw = jnp.dot(h_ref[...], wp, preferred_element_type=jnp.float32)
    d = dinv_ref[...]
    hws_ref[0] = hw[:, :HALF] * d
    hws_ref[1] = hw[:, HALF:] * d


def _scale_call(h, dinv_b, W_gcn, cvec2d):
    return pl.pallas_call(
        _scale_body,
        grid=(N_BLKS,),
        in_specs=[
            pl.BlockSpec((ROWS_BLK, HID), lambda i: (i, 0)),
            pl.BlockSpec((ROWS_BLK, HALF), lambda i: (i, 0)),
            _full((HID, HID)),
            _full((1, HID)),
        ],
        out_specs=pl.BlockSpec((2, ROWS_BLK, HALF), lambda i: (0, i, 0)),
        out_shape=jax.ShapeDtypeStruct((2, N_NODES, HALF), jnp.float32),
    )(h, dinv_b, W_gcn, cvec2d)


def _smooth(h_ref, acc_ref, hwsp_ref, dinv_ref, cvec_ref, bg_ref, bb_ref):
    d = dinv_ref[...]
    a = acc_ref[...]
    p = hwsp_ref[...]
    g0 = (a[0] + p[0]) * d
    g1 = (a[1] + p[1]) * d
    bpp = bg_ref[...] * cvec_ref[...] + bb_ref[...]
    g = jnp.concatenate([g0, g1], axis=1) + bpp
    g = jnp.maximum(g, 0.0)
    return SMOOTH * h_ref[...] + (1.0 - SMOOTH) * g


def _iter_body(h_ref, acc_ref, hwsp_ref, dinv_ref, wg_ref, cvec_ref,
               bg_ref, bb_ref, hn_ref, hws_ref):
    hn = _smooth(h_ref, acc_ref, hwsp_ref, dinv_ref, cvec_ref, bg_ref, bb_ref)
    hn_ref[...] = hn
    wp = wg_ref[...] * cvec_ref[...]
    hw = jnp.dot(hn, wp, preferred_element_type=jnp.float32)
    d = dinv_ref[...]
    hws_ref[0] = hw[:, :HALF] * d
    hws_ref[1] = hw[:, HALF:] * d


def _iter_call(h, acc3, hws3, dinv_b, W_gcn, cvec2d, bg2d, bb2d):
    return pl.pallas_call(
        _iter_body,
        grid=(N_BLKS,),
        in_specs=[
            pl.BlockSpec((ROWS_BLK, HID), lambda i: (i, 0)),
            pl.BlockSpec((2, ROWS_BLK, HALF), lambda i: (0, i, 0)),
            pl.BlockSpec((2, ROWS_BLK, HALF), lambda i: (0, i, 0)),
            pl.BlockSpec((ROWS_BLK, HALF), lambda i: (i, 0)),
            _full((HID, HID)),
            _full((1, HID)),
            _full((1, HID)),
            _full((1, HID)),
        ],
        out_specs=[
            pl.BlockSpec((ROWS_BLK, HID), lambda i: (i, 0)),
            pl.BlockSpec((2, ROWS_BLK, HALF), lambda i: (0, i, 0)),
        ],
        out_shape=[
            jax.ShapeDtypeStruct((N_NODES, HID), jnp.float32),
            jax.ShapeDtypeStruct((2, N_NODES, HALF), jnp.float32),
        ],
    )(h, acc3, hws3, dinv_b, W_gcn, cvec2d, bg2d, bb2d)


def _head_body(h_ref, acc_ref, hwsp_ref, dinv_ref, cvec_ref, bg_ref, bb_ref,
               w1_ref, b1_ref, w2_ref, b2_ref, w3_ref, b3_ref, out_ref):
    hn = _smooth(h_ref, acc_ref, hwsp_ref, dinv_ref, cvec_ref, bg_ref, bb_ref)
    t = jnp.maximum(
        jnp.dot(hn, w1_ref[...], preferred_element_type=jnp.float32)
        + b1_ref[...], 0.0)
    t = jnp.maximum(
        jnp.dot(t, w2_ref[...], preferred_element_type=jnp.float32)
        + b2_ref[...], 0.0)
    out_ref[...] = (
        jnp.dot(t, w3_ref[...], preferred_element_type=jnp.float32)
        + b3_ref[...])


def _head_call(h, acc3, hws3, dinv_b, cvec2d, bg2d, bb2d,
               W1, b1_2d, W2, b2_2d, W3, b3_2d):
    return pl.pallas_call(
        _head_body,
        grid=(N_BLKS,),
        in_specs=[
            pl.BlockSpec((ROWS_BLK, HID), lambda i: (i, 0)),
            pl.BlockSpec((2, ROWS_BLK, HALF), lambda i: (0, i, 0)),
            pl.BlockSpec((2, ROWS_BLK, HALF), lambda i: (0, i, 0)),
            pl.BlockSpec((ROWS_BLK, HALF), lambda i: (i, 0)),
            _full((1, HID)),
            _full((1, HID)),
            _full((1, HID)),
            _full((HID, HID)),
            _full((1, HID)),
            _full((HID, HID)),
            _full((1, HID)),
            _full((HID, OUT_DIM)),
            _full((1, OUT_DIM)),
        ],
        out_specs=pl.BlockSpec((ROWS_BLK, OUT_DIM), lambda i: (i, 0)),
        out_shape=jax.ShapeDtypeStruct((N_NODES, OUT_DIM), jnp.float32),
    )(h, acc3, hws3, dinv_b, cvec2d, bg2d, bb2d, W1, b1_2d, W2, b2_2d, W3, b3_2d)


def kernel(x, edge_index, edge_attr, batch, W_enc, b_enc, W_gcn, b_gcn,
           bn_gamma, bn_beta, W1, b1, W2, b2, W3, b3):
    del edge_attr, batch
    src = edge_index[0]
    dst = edge_index[1]

    inv_std = 1.0 / (1.0 + BN_EPS) ** 0.5
    b_enc2d = b_enc.reshape(1, HID)
    cvec2d = (bn_gamma * inv_std).reshape(1, HID)
    bg2d = b_gcn.reshape(1, HID)
    bb2d = bn_beta.reshape(1, HID)
    b1_2d = b1.reshape(1, HID)
    b2_2d = b2.reshape(1, HID)
    b3_2d = b3.reshape(1, OUT_DIM)

    deg = _deg_kernel(dst)
    deg2d = deg[:N_NODES].reshape(N_BLKS, 1, ROWS_BLK)
    h, dinv2d = _enc_call(x, W_enc, b_enc2d, deg2d)
    dinv_b = jnp.broadcast_to(
        dinv2d.reshape(N_NODES, 1), (N_NODES, HALF))
    hws3 = _scale_call(h, dinv_b, W_gcn, cvec2d)

    for it in range(N_ITERS):
        acc = _agg_kernel(hws3.reshape(2 * N_NODES, HALF), src, dst)
        acc3 = acc.reshape(2, N_NODES, HALF)
        if it < N_ITERS - 1:
            h, hws3 = _iter_call(h, acc3, hws3, dinv_b, W_gcn, cvec2d,
                                 bg2d, bb2d)
        else:
            out = _head_call(h, acc3, hws3, dinv_b, cvec2d, bg2d, bb2d,
                             W1, b1_2d, W2, b2_2d, W3, b3_2d)
    return out


# speedup vs baseline: 13.8245x; 1.1104x over previous
import functools

import jax
import jax.numpy as jnp
from jax import lax
from jax.experimental import pallas as pl
from jax.experimental.pallas import tpu as pltpu
from jax.experimental.pallas import tpu_sc as plsc

N_NODES = 10000
N_EDGES = 160000
IN_DIM = 14
HID = 256
HALF = 128
OUT_DIM = 21
BN_EPS = 1e-5
N_ITERS = 8
SMOOTH = 0.5

ROWS_BLK = 400
N_BLKS = N_NODES // ROWS_BLK
CHUNK = 128
E_PER_SUB = N_EDGES // 16
N_CHUNKS = E_PER_SUB // CHUNK
TAIL = E_PER_SUB - N_CHUNKS * CHUNK
NODE_PAD = 10240
DEG_PER_SUB = NODE_PAD // 16

_sc_mesh = plsc.VectorSubcoreMesh(core_axis_name="c", subcore_axis_name="s")


@functools.partial(
    pl.kernel,
    out_type=jax.ShapeDtypeStruct((NODE_PAD,), jnp.float32),
    mesh=_sc_mesh,
    scratch_types=[
        pltpu.VMEM_SHARED((NODE_PAD,), jnp.float32),
        pltpu.VMEM((CHUNK,), jnp.int32),
        pltpu.VMEM((CHUNK,), jnp.float32),
        pltpu.VMEM((TAIL,), jnp.int32),
        pltpu.VMEM((TAIL,), jnp.float32),
        pltpu.VMEM((DEG_PER_SUB,), jnp.float32),
    ],
)
def _deg_kernel(dst_hbm, deg_hbm, acc_sh, dst_v, ones_v, dst_t, ones_t, buf_v):
    c = lax.axis_index("c")
    s = lax.axis_index("s")

    @pl.when(c == 0)
    def _():
        for t in range(DEG_PER_SUB // 16):
            buf_v[pl.ds(t * 16, 16)] = jnp.zeros((16,), jnp.float32)
        pltpu.sync_copy(buf_v, acc_sh.at[pl.ds(s * DEG_PER_SUB, DEG_PER_SUB)])
        for t in range(CHUNK // 16):
            ones_v[pl.ds(t * 16, 16)] = jnp.full((16,), 1.0, jnp.float32)
        ones_t[...] = jnp.full((TAIL,), 1.0, jnp.float32)
        plsc.subcore_barrier()

        base0 = s * E_PER_SUB

        def body(j, carry):
            base = base0 + j * CHUNK
            pltpu.sync_copy(dst_hbm.at[pl.ds(base, CHUNK)], dst_v)
            pltpu.sync_copy(ones_v, acc_sh.at[dst_v], add=True)
            return carry

        lax.fori_loop(0, N_CHUNKS, body, 0)
        pltpu.sync_copy(dst_hbm.at[pl.ds(base0 + N_CHUNKS * CHUNK, TAIL)], dst_t)
        pltpu.sync_copy(ones_t, acc_sh.at[dst_t], add=True)
        plsc.subcore_barrier()

        pltpu.sync_copy(acc_sh.at[pl.ds(s * DEG_PER_SUB, DEG_PER_SUB)], buf_v)
        for t in range(DEG_PER_SUB // 16):
            sl = pl.ds(t * 16, 16)
            buf_v[sl] = buf_v[sl] + 1.0
        pltpu.sync_copy(buf_v, deg_hbm.at[pl.ds(s * DEG_PER_SUB, DEG_PER_SUB)])


_SUB_STRIDE = 624
_SUB_SPAN = 640

@functools.partial(
    pl.kernel,
    out_type=jax.ShapeDtypeStruct((2 * N_NODES, HALF), jnp.float32),
    mesh=_sc_mesh,
    scratch_types=[
        pltpu.VMEM_SHARED((N_NODES, HALF), jnp.float32),
        pltpu.VMEM((E_PER_SUB,), jnp.int32),
        pltpu.VMEM((CHUNK,), jnp.int32),
        pltpu.VMEM((CHUNK,), jnp.int32),
        pltpu.VMEM((CHUNK, HALF), jnp.float32),
        pltpu.VMEM((CHUNK, HALF), jnp.float32),
        pltpu.SemaphoreType.DMA,
        pltpu.SemaphoreType.DMA,
        pltpu.SemaphoreType.DMA,
        pltpu.SemaphoreType.DMA,
        pltpu.VMEM((TAIL,), jnp.int32),
        pltpu.VMEM((TAIL, HALF), jnp.float32),
        pltpu.SemaphoreType.DMA,
    ],
)
def _agg_kernel(hws_hbm, src_hbm, dst_hbm, out_hbm,
                acc_sh, srcbig, dst0, dst1, rows0, rows1,
                sem0, sem1, dsem0, dsem1, dst_t, rows_t, sem):
    c = lax.axis_index("c")
    s = lax.axis_index("s")
    coff = c * N_NODES
    dst_b = (dst0, dst1)
    rows_b = (rows0, rows1)
    sem_b = (sem0, sem1)
    dsem_b = (dsem0, dsem1)

    base0 = s * E_PER_SUB
    pltpu.sync_copy(src_hbm.at[pl.ds(base0, E_PER_SUB)], srcbig)

    def fzero(k, carry):
        for q in range(HALF // 16):
            rows0[k, pl.ds(q * 16, 16)] = jnp.zeros((16,), jnp.float32)
        return carry

    lax.fori_loop(0, CHUNK, fzero, 0)

    def zbody(k, carry):
        pltpu.sync_copy(rows0,
                        acc_sh.at[pl.ds(s * _SUB_STRIDE + k * CHUNK, CHUNK)])
        return carry

    lax.fori_loop(0, _SUB_SPAN // CHUNK, zbody, 0)

    def cbody(k, carry):
        sl = pl.ds(k * 16, 16)
        srcbig[sl] = srcbig[sl] + coff
        return carry

    lax.fori_loop(0, E_PER_SUB // 16, cbody, 0)
    plsc.subcore_barrier()

    def prefetch(b, j):
        pltpu.async_copy(dst_hbm.at[pl.ds(base0 + j * CHUNK, CHUNK)],
                         dst_b[b], dsem_b[b])
        pltpu.async_copy(hws_hbm.at[srcbig.at[pl.ds(j * CHUNK, CHUNK)]],
                         rows_b[b], sem_b[b])

    prefetch(0, 0)
    prefetch(1, 1)

    def body(k, carry):
        for b in range(2):
            j = 2 * k + b
            pltpu.make_async_copy(
                hws_hbm.at[srcbig.at[pl.ds(j * CHUNK, CHUNK)]],
                rows_b[b], sem_b[b]).wait()
            pltpu.make_async_copy(
                dst_hbm.at[pl.ds(base0 + j * CHUNK, CHUNK)],
                dst_b[b], dsem_b[b]).wait()
            pltpu.sync_copy(rows_b[b], acc_sh.at[dst_b[b]], add=True)

            @pl.when(j + 2 < N_CHUNKS)
            def _():
                prefetch(b, j + 2)

        return carry

    lax.fori_loop(0, N_CHUNKS // 2, body, 0)

    baset = N_CHUNKS * CHUNK
    pltpu.sync_copy(dst_hbm.at[pl.ds(base0 + baset, TAIL)], dst_t)
    pltpu.async_copy(hws_hbm.at[srcbig.at[pl.ds(baset, TAIL)]],
                     rows_t, sem).wait()
    pltpu.sync_copy(rows_t, acc_sh.at[dst_t], add=True)
    plsc.subcore_barrier()

    nbase = s * _SUB_STRIDE
    pltpu.sync_copy(acc_sh.at[pl.ds(nbase, _SUB_SPAN)],
                    out_hbm.at[pl.ds(coff + nbase, _SUB_SPAN)])


def _full(shape):
    return pl.BlockSpec(shape, lambda i: tuple(0 for _ in shape))


def _enc_body(x_ref, we_ref, be_ref, deg_ref, h_ref, dinv_ref):
    dinv_ref[...] = lax.rsqrt(deg_ref[...])
    h_ref[...] = (
        jnp.dot(x_ref[...], we_ref[...], preferred_element_type=jnp.float32)
        + be_ref[...]
    )


def _enc_call(x, W_enc, b_enc2d, deg2d):
    return pl.pallas_call(
        _enc_body,
        grid=(N_BLKS,),
        in_specs=[
            pl.BlockSpec((ROWS_BLK, IN_DIM), lambda i: (i, 0)),
            _full((IN_DIM, HID)),
            _full((1, HID)),
            pl.BlockSpec((1, 1, ROWS_BLK), lambda i: (i, 0, 0)),
        ],
        out_specs=[
            pl.BlockSpec((ROWS_BLK, HID), lambda i: (i, 0)),
            pl.BlockSpec((1, 1, ROWS_BLK), lambda i: (i, 0, 0)),
        ],
        out_shape=[
            jax.ShapeDtypeStruct((N_NODES, HID), jnp.float32),
            jax.ShapeDtypeStruct((N_BLKS, 1, ROWS_BLK), jnp.float32),
        ],
    )(x, W_enc, b_enc2d, deg2d)


def _scale_body(h_ref, dinv_ref, wg_ref, cvec_ref, hws_ref):
    wp = wg_ref[...] * cvec_ref[...]
    hw = jnp.dot(h_ref[...], wp, preferred_element_type=jnp.float32)
    d = dinv_ref[...]
    hws_ref[0] = hw[:, :HALF] * d
    hws_ref[1] = hw[:, HALF:] * d


def _scale_call(h, dinv_b, W_gcn, cvec2d):
    return pl.pallas_call(
        _scale_body,
        grid=(N_BLKS,),
        in_specs=[
            pl.BlockSpec((ROWS_BLK, HID), lambda i: (i, 0)),
            pl.BlockSpec((ROWS_BLK, HALF), lambda i: (i, 0)),
            _full((HID, HID)),
            _full((1, HID)),
        ],
        out_specs=pl.BlockSpec((2, ROWS_BLK, HALF), lambda i: (0, i, 0)),
        out_shape=jax.ShapeDtypeStruct((2, N_NODES, HALF), jnp.float32),
    )(h, dinv_b, W_gcn, cvec2d)


def _smooth(h_ref, acc_ref, hwsp_ref, dinv_ref, cvec_ref, bg_ref, bb_ref):
    d = dinv_ref[...]
    a = acc_ref[...]
    p = hwsp_ref[...]
    g0 = (a[0] + p[0]) * d
    g1 = (a[1] + p[1]) * d
    bpp = bg_ref[...] * cvec_ref[...] + bb_ref[...]
    g = jnp.concatenate([g0, g1], axis=1) + bpp
    g = jnp.maximum(g, 0.0)
    return SMOOTH * h_ref[...] + (1.0 - SMOOTH) * g


def _iter_body(h_ref, acc_ref, hwsp_ref, dinv_ref, wg_ref, cvec_ref,
               bg_ref, bb_ref, hn_ref, hws_ref):
    hn = _smooth(h_ref, acc_ref, hwsp_ref, dinv_ref, cvec_ref, bg_ref, bb_ref)
    hn_ref[...] = hn
    wp = wg_ref[...] * cvec_ref[...]
    hw = jnp.dot(hn, wp, preferred_element_type=jnp.float32)
    d = dinv_ref[...]
    hws_ref[0] = hw[:, :HALF] * d
    hws_ref[1] = hw[:, HALF:] * d


def _iter_call(h, acc3, hws3, dinv_b, W_gcn, cvec2d, bg2d, bb2d):
    return pl.pallas_call(
        _iter_body,
        grid=(N_BLKS,),
        in_specs=[
            pl.BlockSpec((ROWS_BLK, HID), lambda i: (i, 0)),
            pl.BlockSpec((2, ROWS_BLK, HALF), lambda i: (0, i, 0)),
            pl.BlockSpec((2, ROWS_BLK, HALF), lambda i: (0, i, 0)),
            pl.BlockSpec((ROWS_BLK, HALF), lambda i: (i, 0)),
            _full((HID, HID)),
            _full((1, HID)),
            _full((1, HID)),
            _full((1, HID)),
        ],
        out_specs=[
            pl.BlockSpec((ROWS_BLK, HID), lambda i: (i, 0)),
            pl.BlockSpec((2, ROWS_BLK, HALF), lambda i: (0, i, 0)),
        ],
        out_shape=[
            jax.ShapeDtypeStruct((N_NODES, HID), jnp.float32),
            jax.ShapeDtypeStruct((2, N_NODES, HALF), jnp.float32),
        ],
    )(h, acc3, hws3, dinv_b, W_gcn, cvec2d, bg2d, bb2d)


def _head_body(h_ref, acc_ref, hwsp_ref, dinv_ref, cvec_ref, bg_ref, bb_ref,
               w1_ref, b1_ref, w2_ref, b2_ref, w3_ref, b3_ref, out_ref):
    hn = _smooth(h_ref, acc_ref, hwsp_ref, dinv_ref, cvec_ref, bg_ref, bb_ref)
    t = jnp.maximum(
        jnp.dot(hn, w1_ref[...], preferred_element_type=jnp.float32)
        + b1_ref[...], 0.0)
    t = jnp.maximum(
        jnp.dot(t, w2_ref[...], preferred_element_type=jnp.float32)
        + b2_ref[...], 0.0)
    out_ref[...] = (
        jnp.dot(t, w3_ref[...], preferred_element_type=jnp.float32)
        + b3_ref[...])


def _head_call(h, acc3, hws3, dinv_b, cvec2d, bg2d, bb2d,
               W1, b1_2d, W2, b2_2d, W3, b3_2d):
    return pl.pallas_call(
        _head_body,
        grid=(N_BLKS,),
        in_specs=[
            pl.BlockSpec((ROWS_BLK, HID), lambda i: (i, 0)),
            pl.BlockSpec((2, ROWS_BLK, HALF), lambda i: (0, i, 0)),
            pl.BlockSpec((2, ROWS_BLK, HALF), lambda i: (0, i, 0)),
            pl.BlockSpec((ROWS_BLK, HALF), lambda i: (i, 0)),
            _full((1, HID)),
            _full((1, HID)),
            _full((1, HID)),
            _full((HID, HID)),
            _full((1, HID)),
            _full((HID, HID)),
            _full((1, HID)),
            _full((HID, OUT_DIM)),
            _full((1, OUT_DIM)),
        ],
        out_specs=pl.BlockSpec((ROWS_BLK, OUT_DIM), lambda i: (i, 0)),
        out_shape=jax.ShapeDtypeStruct((N_NODES, OUT_DIM), jnp.float32),
    )(h, acc3, hws3, dinv_b, cvec2d, bg2d, bb2d, W1, b1_2d, W2, b2_2d, W3, b3_2d)


def kernel(x, edge_index, edge_attr, batch, W_enc, b_enc, W_gcn, b_gcn,
           bn_gamma, bn_beta, W1, b1, W2, b2, W3, b3):
    del edge_attr, batch
    src = edge_index[0]
    dst = edge_index[1]

    inv_std = 1.0 / (1.0 + BN_EPS) ** 0.5
    b_enc2d = b_enc.reshape(1, HID)
    cvec2d = (bn_gamma * inv_std).reshape(1, HID)
    bg2d = b_gcn.reshape(1, HID)
    bb2d = bn_beta.reshape(1, HID)
    b1_2d = b1.reshape(1, HID)
    b2_2d = b2.reshape(1, HID)
    b3_2d = b3.reshape(1, OUT_DIM)

    deg = _deg_kernel(dst)
    deg2d = deg[:N_NODES].reshape(N_BLKS, 1, ROWS_BLK)
    h, dinv2d = _enc_call(x, W_enc, b_enc2d, deg2d)
    dinv_b = jnp.broadcast_to(
        dinv2d.reshape(N_NODES, 1), (N_NODES, HALF))
    hws3 = _scale_call(h, dinv_b, W_gcn, cvec2d)

    for it in range(N_ITERS):
        acc = _agg_kernel(hws3.reshape(2 * N_NODES, HALF), src, dst)
        acc3 = acc.reshape(2, N_NODES, HALF)
        if it < N_ITERS - 1:
            h, hws3 = _iter_call(h, acc3, hws3, dinv_b, W_gcn, cvec2d,
                                 bg2d, bb2d)
        else:
            out = _head_call(h, acc3, hws3, dinv_b, cvec2d, bg2d, bb2d,
                             W1, b1_2d, W2, b2_2d, W3, b3_2d)
    return out


# speedup vs baseline: 14.0723x; 1.0179x over previous
import functools

import jax
import jax.numpy as jnp
from jax import lax
from jax.experimental import pallas as pl
from jax.experimental.pallas import tpu as pltpu
from jax.experimental.pallas import tpu_sc as plsc

N_NODES = 10000
N_EDGES = 160000
IN_DIM = 14
HID = 256
HALF = 128
OUT_DIM = 21
BN_EPS = 1e-5
N_ITERS = 8
SMOOTH = 0.5

ROWS_BLK = 400
N_BLKS = N_NODES // ROWS_BLK
CHUNK = 128
E_PER_SUB = N_EDGES // 16
N_CHUNKS = E_PER_SUB // CHUNK
TAIL = E_PER_SUB - N_CHUNKS * CHUNK
NODE_PAD = 10240
DEG_PER_SUB = NODE_PAD // 16

_sc_mesh = plsc.VectorSubcoreMesh(core_axis_name="c", subcore_axis_name="s")


@functools.partial(
    pl.kernel,
    out_type=jax.ShapeDtypeStruct((NODE_PAD,), jnp.float32),
    mesh=_sc_mesh,
    scratch_types=[
        pltpu.VMEM_SHARED((NODE_PAD,), jnp.float32),
        pltpu.VMEM((CHUNK,), jnp.int32),
        pltpu.VMEM((CHUNK,), jnp.int32),
        pltpu.SemaphoreType.DMA,
        pltpu.SemaphoreType.DMA,
        pltpu.VMEM((CHUNK,), jnp.float32),
        pltpu.VMEM((TAIL,), jnp.int32),
        pltpu.VMEM((TAIL,), jnp.float32),
        pltpu.VMEM((DEG_PER_SUB,), jnp.float32),
    ],
)
def _deg_kernel(dst_hbm, deg_hbm, acc_sh, dstA, dstB, semA, semB,
                ones_v, dst_t, ones_t, buf_v):
    c = lax.axis_index("c")
    s = lax.axis_index("s")

    @pl.when(c == 0)
    def _():
        for t in range(DEG_PER_SUB // 16):
            buf_v[pl.ds(t * 16, 16)] = jnp.zeros((16,), jnp.float32)
        pltpu.sync_copy(buf_v, acc_sh.at[pl.ds(s * DEG_PER_SUB, DEG_PER_SUB)])
        for t in range(CHUNK // 16):
            ones_v[pl.ds(t * 16, 16)] = jnp.full((16,), 1.0, jnp.float32)
        ones_t[...] = jnp.full((TAIL,), 1.0, jnp.float32)
        plsc.subcore_barrier()

        base0 = s * E_PER_SUB
        dst_b = (dstA, dstB)
        sem_b = (semA, semB)

        def prefetch(b, j):
            pltpu.async_copy(dst_hbm.at[pl.ds(base0 + j * CHUNK, CHUNK)],
                             dst_b[b], sem_b[b])

        prefetch(0, 0)
        prefetch(1, 1)

        def body(k, carry):
            for b in range(2):
                j = 2 * k + b
                pltpu.make_async_copy(
                    dst_hbm.at[pl.ds(base0 + j * CHUNK, CHUNK)],
                    dst_b[b], sem_b[b]).wait()
                pltpu.sync_copy(ones_v, acc_sh.at[dst_b[b]], add=True)

                @pl.when(j + 2 < N_CHUNKS)
                def _():
                    prefetch(b, j + 2)

            return carry

        lax.fori_loop(0, N_CHUNKS // 2, body, 0)
        pltpu.sync_copy(dst_hbm.at[pl.ds(base0 + N_CHUNKS * CHUNK, TAIL)], dst_t)
        pltpu.sync_copy(ones_t, acc_sh.at[dst_t], add=True)
        plsc.subcore_barrier()

        pltpu.sync_copy(acc_sh.at[pl.ds(s * DEG_PER_SUB, DEG_PER_SUB)], buf_v)
        for t in range(DEG_PER_SUB // 16):
            sl = pl.ds(t * 16, 16)
            buf_v[sl] = buf_v[sl] + 1.0
        pltpu.sync_copy(buf_v, deg_hbm.at[pl.ds(s * DEG_PER_SUB, DEG_PER_SUB)])


_SUB_STRIDE = 624
_SUB_SPAN = 640

@functools.partial(
    pl.kernel,
    out_type=jax.ShapeDtypeStruct((2 * N_NODES, HALF), jnp.float32),
    mesh=_sc_mesh,
    scratch_types=[
        pltpu.VMEM_SHARED((N_NODES, HALF), jnp.float32),
        pltpu.VMEM((E_PER_SUB,), jnp.int32),
        pltpu.VMEM((CHUNK,), jnp.int32),
        pltpu.VMEM((CHUNK,), jnp.int32),
        pltpu.VMEM((CHUNK, HALF), jnp.float32),
        pltpu.VMEM((CHUNK, HALF), jnp.float32),
        pltpu.SemaphoreType.DMA,
        pltpu.SemaphoreType.DMA,
        pltpu.SemaphoreType.DMA,
        pltpu.SemaphoreType.DMA,
        pltpu.VMEM((TAIL,), jnp.int32),
        pltpu.VMEM((TAIL, HALF), jnp.float32),
        pltpu.SemaphoreType.DMA,
    ],
)
def _agg_kernel(hws_hbm, src_hbm, dst_hbm, out_hbm,
                acc_sh, srcbig, dst0, dst1, rows0, rows1,
                sem0, sem1, dsem0, dsem1, dst_t, rows_t, sem):
    c = lax.axis_index("c")
    s = lax.axis_index("s")
    coff = c * N_NODES
    dst_b = (dst0, dst1)
    rows_b = (rows0, rows1)
    sem_b = (sem0, sem1)
    dsem_b = (dsem0, dsem1)

    base0 = s * E_PER_SUB
    pltpu.sync_copy(src_hbm.at[pl.ds(base0, E_PER_SUB)], srcbig)

    def fzero(k, carry):
        for q in range(HALF // 16):
            rows0[k, pl.ds(q * 16, 16)] = jnp.zeros((16,), jnp.float32)
        return carry

    lax.fori_loop(0, CHUNK, fzero, 0)

    def zbody(k, carry):
        pltpu.sync_copy(rows0,
                        acc_sh.at[pl.ds(s * _SUB_STRIDE + k * CHUNK, CHUNK)])
        return carry

    lax.fori_loop(0, _SUB_SPAN // CHUNK, zbody, 0)

    def cbody(k, carry):
        sl = pl.ds(k * 16, 16)
        srcbig[sl] = srcbig[sl] + coff
        return carry

    lax.fori_loop(0, E_PER_SUB // 16, cbody, 0)
    plsc.subcore_barrier()

    def prefetch(b, j):
        pltpu.async_copy(dst_hbm.at[pl.ds(base0 + j * CHUNK, CHUNK)],
                         dst_b[b], dsem_b[b])
        pltpu.async_copy(hws_hbm.at[srcbig.at[pl.ds(j * CHUNK, CHUNK)]],
                         rows_b[b], sem_b[b])

    prefetch(0, 0)
    prefetch(1, 1)

    def body(k, carry):
        for b in range(2):
            j = 2 * k + b
            pltpu.make_async_copy(
                hws_hbm.at[srcbig.at[pl.ds(j * CHUNK, CHUNK)]],
                rows_b[b], sem_b[b]).wait()
            pltpu.make_async_copy(
                dst_hbm.at[pl.ds(base0 + j * CHUNK, CHUNK)],
                dst_b[b], dsem_b[b]).wait()
            pltpu.sync_copy(rows_b[b], acc_sh.at[dst_b[b]], add=True)

            @pl.when(j + 2 < N_CHUNKS)
            def _():
                prefetch(b, j + 2)

        return carry

    lax.fori_loop(0, N_CHUNKS // 2, body, 0)

    baset = N_CHUNKS * CHUNK
    pltpu.sync_copy(dst_hbm.at[pl.ds(base0 + baset, TAIL)], dst_t)
    pltpu.async_copy(hws_hbm.at[srcbig.at[pl.ds(baset, TAIL)]],
                     rows_t, sem).wait()
    pltpu.sync_copy(rows_t, acc_sh.at[dst_t], add=True)
    plsc.subcore_barrier()

    nbase = s * _SUB_STRIDE
    pltpu.sync_copy(acc_sh.at[pl.ds(nbase, _SUB_SPAN)],
                    out_hbm.at[pl.ds(coff + nbase, _SUB_SPAN)])


def _full(shape):
    return pl.BlockSpec(shape, lambda i: tuple(0 for _ in shape))


def _enc_body(x_ref, we_ref, be_ref, deg_ref, h_ref, dinv_ref):
    dinv_ref[...] = lax.rsqrt(deg_ref[...])
    h_ref[...] = (
        jnp.dot(x_ref[...], we_ref[...], preferred_element_type=jnp.float32)
        + be_ref[...]
    )


def _enc_call(x, W_enc, b_enc2d, deg2d):
    return pl.pallas_call(
        _enc_body,
        grid=(N_BLKS,),
        in_specs=[
            pl.BlockSpec((ROWS_BLK, IN_DIM), lambda i: (i, 0)),
            _full((IN_DIM, HID)),
            _full((1, HID)),
            pl.BlockSpec((1, 1, ROWS_BLK), lambda i: (i, 0, 0)),
        ],
        out_specs=[
            pl.BlockSpec((ROWS_BLK, HID), lambda i: (i, 0)),
            pl.BlockSpec((1, 1, ROWS_BLK), lambda i: (i, 0, 0)),
        ],
        out_shape=[
            jax.ShapeDtypeStruct((N_NODES, HID), jnp.float32),
            jax.ShapeDtypeStruct((N_BLKS, 1, ROWS_BLK), jnp.float32),
        ],
    )(x, W_enc, b_enc2d, deg2d)


def _scale_body(h_ref, dinv_ref, wg_ref, cvec_ref, hws_ref):
    wp = wg_ref[...] * cvec_ref[...]
    hw = jnp.dot(h_ref[...], wp, preferred_element_type=jnp.float32)
    d = dinv_ref[...]
    hws_ref[0] = hw[:, :HALF] * d
    hws_ref[1] = hw[:, HALF:] * d


def _scale_call(h, dinv_b, W_gcn, cvec2d):
    return pl.pallas_call(
        _scale_body,
        grid=(N_BLKS,),
        in_specs=[
            pl.BlockSpec((ROWS_BLK, HID), lambda i: (i, 0)),
            pl.BlockSpec((ROWS_BLK, HALF), lambda i: (i, 0)),
            _full((HID, HID)),
            _full((1, HID)),
        ],
        out_specs=pl.BlockSpec((2, ROWS_BLK, HALF), lambda i: (0, i, 0)),
        out_shape=jax.ShapeDtypeStruct((2, N_NODES, HALF), jnp.float32),
    )(h, dinv_b, W_gcn, cvec2d)


def _smooth(h_ref, acc_ref, hwsp_ref, dinv_ref, cvec_ref, bg_ref, bb_ref):
    d = dinv_ref[...]
    a = acc_ref[...]
    p = hwsp_ref[...]
    g0 = (a[0] + p[0]) * d
    g1 = (a[1] + p[1]) * d
    bpp = bg_ref[...] * cvec_ref[...] + bb_ref[...]
    g = jnp.concatenate([g0, g1], axis=1) + bpp
    g = jnp.maximum(g, 0.0)
    return SMOOTH * h_ref[...] + (1.0 - SMOOTH) * g


def _iter_body(h_ref, acc_ref, hwsp_ref, dinv_ref, wg_ref, cvec_ref,
               bg_ref, bb_ref, hn_ref, hws_ref):
    hn = _smooth(h_ref, acc_ref, hwsp_ref, dinv_ref, cvec_ref, bg_ref, bb_ref)
    hn_ref[...] = hn
    wp = wg_ref[...] * cvec_ref[...]
    hw = jnp.dot(hn, wp, preferred_element_type=jnp.float32)
    d = dinv_ref[...]
    hws_ref[0] = hw[:, :HALF] * d
    hws_ref[1] = hw[:, HALF:] * d


def _iter_call(h, acc3, hws3, dinv_b, W_gcn, cvec2d, bg2d, bb2d):
    return pl.pallas_call(
        _iter_body,
        grid=(N_BLKS,),
        in_specs=[
            pl.BlockSpec((ROWS_BLK, HID), lambda i: (i, 0)),
            pl.BlockSpec((2, ROWS_BLK, HALF), lambda i: (0, i, 0)),
            pl.BlockSpec((2, ROWS_BLK, HALF), lambda i: (0, i, 0)),
            pl.BlockSpec((ROWS_BLK, HALF), lambda i: (i, 0)),
            _full((HID, HID)),
            _full((1, HID)),
            _full((1, HID)),
            _full((1, HID)),
        ],
        out_specs=[
            pl.BlockSpec((ROWS_BLK, HID), lambda i: (i, 0)),
            pl.BlockSpec((2, ROWS_BLK, HALF), lambda i: (0, i, 0)),
        ],
        out_shape=[
            jax.ShapeDtypeStruct((N_NODES, HID), jnp.float32),
            jax.ShapeDtypeStruct((2, N_NODES, HALF), jnp.float32),
        ],
    )(h, acc3, hws3, dinv_b, W_gcn, cvec2d, bg2d, bb2d)


def _head_body(h_ref, acc_ref, hwsp_ref, dinv_ref, cvec_ref, bg_ref, bb_ref,
               w1_ref, b1_ref, w2_ref, b2_ref, w3_ref, b3_ref, out_ref):
    hn = _smooth(h_ref, acc_ref, hwsp_ref, dinv_ref, cvec_ref, bg_ref, bb_ref)
    t = jnp.maximum(
        jnp.dot(hn, w1_ref[...], preferred_element_type=jnp.float32)
        + b1_ref[...], 0.0)
    t = jnp.maximum(
        jnp.dot(t, w2_ref[...], preferred_element_type=jnp.float32)
        + b2_ref[...], 0.0)
    out_ref[...] = (
        jnp.dot(t, w3_ref[...], preferred_element_type=jnp.float32)
        + b3_ref[...])


def _head_call(h, acc3, hws3, dinv_b, cvec2d, bg2d, bb2d,
               W1, b1_2d, W2, b2_2d, W3, b3_2d):
    return pl.pallas_call(
        _head_body,
        grid=(N_BLKS,),
        in_specs=[
            pl.BlockSpec((ROWS_BLK, HID), lambda i: (i, 0)),
            pl.BlockSpec((2, ROWS_BLK, HALF), lambda i: (0, i, 0)),
            pl.BlockSpec((2, ROWS_BLK, HALF), lambda i: (0, i, 0)),
            pl.BlockSpec((ROWS_BLK, HALF), lambda i: (i, 0)),
            _full((1, HID)),
            _full((1, HID)),
            _full((1, HID)),
            _full((HID, HID)),
            _full((1, HID)),
            _full((HID, HID)),
            _full((1, HID)),
            _full((HID, OUT_DIM)),
            _full((1, OUT_DIM)),
        ],
        out_specs=pl.BlockSpec((ROWS_BLK, OUT_DIM), lambda i: (i, 0)),
        out_shape=jax.ShapeDtypeStruct((N_NODES, OUT_DIM), jnp.float32),
    )(h, acc3, hws3, dinv_b, cvec2d, bg2d, bb2d, W1, b1_2d, W2, b2_2d, W3, b3_2d)


def kernel(x, edge_index, edge_attr, batch, W_enc, b_enc, W_gcn, b_gcn,
           bn_gamma, bn_beta, W1, b1, W2, b2, W3, b3):
    del edge_attr, batch
    src = edge_index[0]
    dst = edge_index[1]

    inv_std = 1.0 / (1.0 + BN_EPS) ** 0.5
    b_enc2d = b_enc.reshape(1, HID)
    cvec2d = (bn_gamma * inv_std).reshape(1, HID)
    bg2d = b_gcn.reshape(1, HID)
    bb2d = bn_beta.reshape(1, HID)
    b1_2d = b1.reshape(1, HID)
    b2_2d = b2.reshape(1, HID)
    b3_2d = b3.reshape(1, OUT_DIM)

    deg = _deg_kernel(dst)
    deg2d = deg[:N_NODES].reshape(N_BLKS, 1, ROWS_BLK)
    h, dinv2d = _enc_call(x, W_enc, b_enc2d, deg2d)
    dinv_b = jnp.broadcast_to(
        dinv2d.reshape(N_NODES, 1), (N_NODES, HALF))
    hws3 = _scale_call(h, dinv_b, W_gcn, cvec2d)

    for it in range(N_ITERS):
        acc = _agg_kernel(hws3.reshape(2 * N_NODES, HALF), src, dst)
        acc3 = acc.reshape(2, N_NODES, HALF)
        if it < N_ITERS - 1:
            h, hws3 = _iter_call(h, acc3, hws3, dinv_b, W_gcn, cvec2d,
                                 bg2d, bb2d)
        else:
            out = _head_call(h, acc3, hws3, dinv_b, cvec2d, bg2d, bb2d,
                             W1, b1_2d, W2, b2_2d, W3, b3_2d)
    return out
